# baseline (speedup 1.0000x reference)
import functools

import jax
import jax.numpy as jnp
from jax import lax
from jax.experimental import pallas as pl
from jax.experimental.pallas import tpu as pltpu
from jax.experimental.pallas import tpu_sc as plsc

B = 1024
N_CAND = 50000
N_FEAT = 96
D_MAIN = 128
D_BLOCK = 256
CTX = 96

NC_PAD = 50176
N_CHUNK = NC_PAD // 128
CMIN_PAD = 512
AUG = 144
I32_MAX = 2147483647


def _enc_cand_body(x_ref, wi_ref, bi_ref, wk_ref, bk_ref, tbl_ref, aug_ref):
    i = pl.program_id(0)
    cx = jnp.dot(x_ref[:], wi_ref[:], preferred_element_type=jnp.float32) + bi_ref[:]
    ck = jnp.dot(cx, wk_ref[:], preferred_element_type=jnp.float32) + bk_ref[:]
    tbl_ref[:] = ck
    rows = i * 512 + lax.broadcasted_iota(jnp.int32, (512, 1), 0)
    pad = rows >= N_CAND
    cksq = jnp.sum(ck * ck, axis=-1, keepdims=True)
    cksq = jnp.where(pad, 1e30, cksq)
    one = jnp.ones((512, 1), jnp.float32)
    zero = jnp.zeros((512, AUG - 130), jnp.float32)
    aug_ref[:] = jnp.concatenate([ck, cksq, one, zero], axis=1)


def _encode_candidates(cand_p, W_in, b_in, W_K, b_K):
    return pl.pallas_call(
        _enc_cand_body,
        grid=(NC_PAD // 512,),
        in_specs=[
            pl.BlockSpec((512, N_FEAT), lambda i: (i, 0)),
            pl.BlockSpec((N_FEAT, D_MAIN), lambda i: (0, 0)),
            pl.BlockSpec((1, D_MAIN), lambda i: (0, 0)),
            pl.BlockSpec((D_MAIN, D_MAIN), lambda i: (0, 0)),
            pl.BlockSpec((1, D_MAIN), lambda i: (0, 0)),
        ],
        out_specs=[
            pl.BlockSpec((512, D_MAIN), lambda i: (i, 0)),
            pl.BlockSpec((512, AUG), lambda i: (i, 0)),
        ],
        out_shape=[
            jax.ShapeDtypeStruct((NC_PAD, D_MAIN), jnp.float32),
            jax.ShapeDtypeStruct((NC_PAD, AUG), jnp.float32),
        ],
    )(cand_p, W_in, b_in, W_K, b_K)


def _enc_query_body(x_ref, wi_ref, bi_ref, wk_ref, bk_ref, x1_ref, k_ref, kaug_ref):
    x1 = jnp.dot(x_ref[:], wi_ref[:], preferred_element_type=jnp.float32) + bi_ref[:]
    k = jnp.dot(x1, wk_ref[:], preferred_element_type=jnp.float32) + bk_ref[:]
    x1_ref[:] = x1
    k_ref[:] = k
    ksq = jnp.sum(k * k, axis=-1, keepdims=True)
    one = jnp.ones((256, 1), jnp.float32)
    zero = jnp.zeros((256, AUG - 130), jnp.float32)
    kaug_ref[:] = jnp.concatenate([-2.0 * k, one, ksq, zero], axis=1)


def _encode_queries(x_num, W_in, b_in, W_K, b_K):
    return pl.pallas_call(
        _enc_query_body,
        grid=(B // 256,),
        in_specs=[
            pl.BlockSpec((256, N_FEAT), lambda i: (i, 0)),
            pl.BlockSpec((N_FEAT, D_MAIN), lambda i: (0, 0)),
            pl.BlockSpec((1, D_MAIN), lambda i: (0, 0)),
            pl.BlockSpec((D_MAIN, D_MAIN), lambda i: (0, 0)),
            pl.BlockSpec((1, D_MAIN), lambda i: (0, 0)),
        ],
        out_specs=[
            pl.BlockSpec((256, D_MAIN), lambda i: (i, 0)),
            pl.BlockSpec((256, D_MAIN), lambda i: (i, 0)),
            pl.BlockSpec((256, AUG), lambda i: (i, 0)),
        ],
        out_shape=[
            jax.ShapeDtypeStruct((B, D_MAIN), jnp.float32),
            jax.ShapeDtypeStruct((B, D_MAIN), jnp.float32),
            jax.ShapeDtypeStruct((B, AUG), jnp.float32),
        ],
    )(x_num, W_in, b_in, W_K, b_K)


def _dist_body(kaug_ref, caug_ref, keys_ref):
    d2 = lax.dot_general(kaug_ref[:], caug_ref[:], (((1,), (1,)), ((), ())),
                         preferred_element_type=jnp.float32)
    u = lax.bitcast_convert_type(d2, jnp.int32)
    keys_ref[:] = jnp.where(u < 0, u ^ 0x7FFFFFFF, u)


def _distance_keys(k_aug, ck_aug):
    RB, CB = 256, 6272
    return pl.pallas_call(
        _dist_body,
        grid=(B // RB, NC_PAD // CB),
        in_specs=[
            pl.BlockSpec((RB, AUG), lambda i, j: (i, 0)),
            pl.BlockSpec((CB, AUG), lambda i, j: (j, 0)),
        ],
        out_specs=pl.BlockSpec((RB, CB), lambda i, j: (i, j)),
        out_shape=jax.ShapeDtypeStruct((B, NC_PAD), jnp.int32),
    )(k_aug, ck_aug)


def _thresh_body(keys_ref, t_ref):
    keys = keys_ref[:]
    rb = keys.shape[0]
    lo0 = jnp.full((rb, 1), -2147483648, jnp.int32)
    hi0 = jnp.full((rb, 1), I32_MAX, jnp.int32)

    def body(_, carry):
        lo, hi = carry
        mid = (lo & hi) + ((lo ^ hi) >> 1)
        cnt = jnp.sum((keys <= mid).astype(jnp.int32), axis=1, keepdims=True)
        ge = cnt >= CTX
        return jnp.where(ge, lo, mid), jnp.where(ge, mid, hi)

    lo, hi = lax.fori_loop(0, 32, body, (lo0, hi0))
    t_ref[:] = jnp.broadcast_to(hi, (rb, 128))


def _threshold(keys):
    RB = 64
    return pl.pallas_call(
        _thresh_body,
        grid=(B // RB,),
        in_specs=[pl.BlockSpec((RB, NC_PAD), lambda i: (i, 0))],
        out_specs=pl.BlockSpec((RB, 128), lambda i: (i, 0)),
        out_shape=jax.ShapeDtypeStruct((B, 128), jnp.int32),
    )(keys)


NV = NC_PAD // 16
ROWS_PER_TILE = B // 32


def _sc_body(keys_hbm, t_hbm, ck_hbm, y_hbm, ctx_hbm, ysel_hbm,
             keys_v, y_v, gath_v, t_v, lt_v, eq_v, idx_v, ys_v, sem):
    wid = lax.axis_index("s") * 2 + lax.axis_index("c")
    pltpu.sync_copy(y_hbm, y_v)
    iota16 = lax.iota(jnp.int32, 16)

    zeros16i = jnp.zeros((16,), jnp.int32)

    def row_body(r, _):
        row = wid * ROWS_PER_TILE + r
        pltpu.sync_copy(keys_hbm.at[row], keys_v)
        pltpu.sync_copy(t_hbm.at[row], t_v)
        tvec = t_v[pl.ds(0, 16)]

        def group_body(g, c_le):
            cc = c_le
            for u in range(8):
                v = keys_v[pl.ds((g * 8 + u) * 16, 16)]
                le = v <= tvec
                ivec = (g * 8 + u) * 16 + iota16
                pos = cc + plsc.cumsum(le.astype(jnp.int32)) - 1
                plsc.store_scatter(idx_v, [pos], ivec,
                                   mask=le & (pos < CTX))
                cc = cc + plsc.all_reduce_population_count(le)
            return cc

        c_le = lax.fori_loop(0, NV // 8, group_body, zeros16i)

        def slow(_):
            def vreg_body(j, carry):
                cnt_lt, cnt_eq = carry
                v = keys_v[pl.ds(j * 16, 16)]
                le = v <= tvec

                def hit2(c):
                    c_lt, c_eq = c
                    lt = v < tvec
                    eq = le & (~lt)
                    ivec = j * 16 + iota16
                    pos_lt = c_lt + plsc.cumsum(lt.astype(jnp.int32)) - 1
                    pos_eq = c_eq + plsc.cumsum(eq.astype(jnp.int32)) - 1
                    plsc.store_scatter(lt_v, [pos_lt], ivec,
                                       mask=lt & (pos_lt < CTX))
                    plsc.store_scatter(eq_v, [pos_eq], ivec,
                                       mask=eq & (pos_eq < CTX))
                    return (c_lt + plsc.all_reduce_population_count(lt),
                            c_eq + plsc.all_reduce_population_count(eq))

                return lax.cond(jnp.any(le), hit2, lambda c: c,
                                (cnt_lt, cnt_eq))

            clt, _ceq = lax.fori_loop(0, NV, vreg_body,
                                      (zeros16i, zeros16i))
            for jv in range(CTX // 16):
                jvec = jv * 16 + iota16
                a = plsc.load_gather(lt_v, [jnp.minimum(jvec, CTX - 1)])
                b = plsc.load_gather(eq_v, [jnp.clip(jvec - clt, 0, CTX - 1)])
                sel = jnp.where(jvec < clt, a, b)
                idx_v[pl.ds(jv * 16, 16)] = sel
            return 0

        lax.cond(jnp.any(c_le != CTX), slow, lambda _: 0, 0)

        for jv in range(CTX // 16):
            sel = idx_v[pl.ds(jv * 16, 16)]
            ys_v[pl.ds(jv * 16, 16)] = plsc.load_gather(y_v, [sel])
        zeros16 = jnp.zeros((16,), jnp.float32)
        ys_v[pl.ds(96, 16)] = zeros16
        ys_v[pl.ds(112, 16)] = zeros16
        pltpu.async_copy(ck_hbm.at[idx_v], gath_v, sem).wait()
        pltpu.sync_copy(gath_v, ctx_hbm.at[pl.ds(row * CTX, CTX)])
        pltpu.sync_copy(ys_v, ysel_hbm.at[row])
        return 0

    lax.fori_loop(0, ROWS_PER_TILE, row_body, 0)


def _sc_select_gather(keys, t_bcast, ck_tbl, y_p):
    f = functools.partial(
        pl.kernel,
        mesh=plsc.VectorSubcoreMesh(core_axis_name="c", subcore_axis_name="s"),
        compiler_params=pltpu.CompilerParams(needs_layout_passes=False),
        out_type=[
            jax.ShapeDtypeStruct((B * CTX, D_MAIN), jnp.float32),
            jax.ShapeDtypeStruct((B, 128), jnp.float32),
        ],
        scratch_types=[
            pltpu.VMEM((NC_PAD,), jnp.int32),
            pltpu.VMEM((NC_PAD,), jnp.float32),
            pltpu.VMEM((CTX, D_MAIN), jnp.float32),
            pltpu.VMEM((128,), jnp.int32),
            pltpu.VMEM((CTX,), jnp.int32),
            pltpu.VMEM((CTX,), jnp.int32),
            pltpu.VMEM((CTX,), jnp.int32),
            pltpu.VMEM((128,), jnp.float32),
            pltpu.SemaphoreType.DMA,
        ],
    )(_sc_body)
    return f(keys, t_bcast, ck_tbl, y_p)


def _final_body(x1_ref, k_ref, ctx_ref, ys_ref,
                wle_ref, ble_ref, wt1_ref, bt1_ref, wt2_ref,
                g1_ref, bb1_ref, wb1_ref, b_b1_ref, wb2_ref, b_b2_ref,
                hg_ref, hb_ref, wh_ref, bh_ref, out_ref):
    rb = x1_ref.shape[0]
    k = k_ref[:]
    ksq = jnp.sum(k * k, axis=-1, keepdims=True)
    ck3 = ctx_ref[:].reshape(rb, CTX, D_MAIN)
    kb = k[:, None, :]
    sims = (-ksq + 2.0 * jnp.sum(kb * ck3, axis=-1)
            - jnp.sum(ck3 * ck3, axis=-1))
    m = jnp.max(sims, axis=-1, keepdims=True)
    e = jnp.exp(sims - m)
    probs = e / jnp.sum(e, axis=-1, keepdims=True)

    t_in = (kb - ck3).reshape(rb * CTX, D_MAIN)
    hid = jnp.maximum(
        jnp.dot(t_in, wt1_ref[:], preferred_element_type=jnp.float32) + bt1_ref[:], 0.0)
    v = jnp.dot(hid, wt2_ref[:], preferred_element_type=jnp.float32)
    w3 = wle_ref[:].reshape(1, 1, D_MAIN)
    b3 = ble_ref[:].reshape(1, 1, D_MAIN)
    v3 = v.reshape(rb, CTX, D_MAIN) + ys_ref[:][:, :, None] * w3 + b3
    ctxx = jnp.sum(probs[:, :, None] * v3, axis=1)
    x = x1_ref[:] + ctxx

    mu = x.mean(-1, keepdims=True)
    var = ((x - mu) ** 2).mean(-1, keepdims=True)
    h = (x - mu) / jnp.sqrt(var + 1e-5) * g1_ref[:] + bb1_ref[:]
    x = x + (jnp.maximum(
        jnp.dot(h, wb1_ref[:], preferred_element_type=jnp.float32) + b_b1_ref[:], 0.0)
        @ wb2_ref[:] + b_b2_ref[:])
    mu = x.mean(-1, keepdims=True)
    var = ((x - mu) ** 2).mean(-1, keepdims=True)
    h = (x - mu) / jnp.sqrt(var + 1e-5) * hg_ref[:] + hb_ref[:]
    out = jnp.dot(jnp.maximum(h, 0.0), wh_ref[:],
                  preferred_element_type=jnp.float32) + bh_ref[:]
    out_ref[:] = out


def _final(x1, k, ctx_rows, ysel, W_le, b_le, W_T1, b_T1, W_T2,
           ln1_g, ln1_b, W_b1, b_b1, W_b2, b_b2, h_g, h_b, W_h, b_h):
    RB = 128
    r2 = lambda a: a.reshape(1, -1)
    return pl.pallas_call(
        _final_body,
        grid=(B // RB,),
        in_specs=[
            pl.BlockSpec((RB, D_MAIN), lambda i: (i, 0)),
            pl.BlockSpec((RB, D_MAIN), lambda i: (i, 0)),
            pl.BlockSpec((RB * CTX, D_MAIN), lambda i: (i, 0)),
            pl.BlockSpec((RB, CTX), lambda i: (i, 0)),
            pl.BlockSpec((1, D_MAIN), lambda i: (0, 0)),
            pl.BlockSpec((1, D_MAIN), lambda i: (0, 0)),
            pl.BlockSpec((D_MAIN, D_BLOCK), lambda i: (0, 0)),
            pl.BlockSpec((1, D_BLOCK), lambda i: (0, 0)),
            pl.BlockSpec((D_BLOCK, D_MAIN), lambda i: (0, 0)),
            pl.BlockSpec((1, D_MAIN), lambda i: (0, 0)),
            pl.BlockSpec((1, D_MAIN), lambda i: (0, 0)),
            pl.BlockSpec((D_MAIN, D_BLOCK), lambda i: (0, 0)),
            pl.BlockSpec((1, D_BLOCK), lambda i: (0, 0)),
            pl.BlockSpec((D_BLOCK, D_MAIN), lambda i: (0, 0)),
            pl.BlockSpec((1, D_MAIN), lambda i: (0, 0)),
            pl.BlockSpec((1, D_MAIN), lambda i: (0, 0)),
            pl.BlockSpec((1, D_MAIN), lambda i: (0, 0)),
            pl.BlockSpec((D_MAIN, 1), lambda i: (0, 0)),
            pl.BlockSpec((1, 1), lambda i: (0, 0)),
        ],
        out_specs=pl.BlockSpec((RB, 1), lambda i: (i, 0)),
        out_shape=jax.ShapeDtypeStruct((B, 1), jnp.float32),
    )(x1, k, ctx_rows, ysel, r2(W_le), r2(b_le), W_T1, r2(b_T1), W_T2,
      r2(ln1_g), r2(ln1_b), W_b1, r2(b_b1), W_b2, r2(b_b2), r2(h_g), r2(h_b),
      W_h, b_h.reshape(1, 1))


def kernel(x_num, candidate_x_num, candidate_y, W_in, b_in, W_K, b_K, W_le, b_le,
           W_T1, b_T1, W_T2, ln1_g, ln1_b, W_b1, b_b1, W_b2, b_b2, h_g, h_b, W_h, b_h,
           context_size):
    cand_p = jnp.pad(candidate_x_num, ((0, NC_PAD - N_CAND), (0, 0)))
    y_p = jnp.pad(candidate_y, (0, NC_PAD - N_CAND))
    bi = b_in.reshape(1, -1)
    bk = b_K.reshape(1, -1)

    ck_tbl, ck_aug = _encode_candidates(cand_p, W_in, bi, W_K, bk)
    x1, k, k_aug = _encode_queries(x_num, W_in, bi, W_K, bk)
    keys = _distance_keys(k_aug, ck_aug)
    t_bcast = _threshold(keys)

    ctx_rows, ysel_pad = _sc_select_gather(keys, t_bcast, ck_tbl, y_p)
    ysel = ysel_pad[:, :CTX]

    out = _final(x1, k, ctx_rows, ysel, W_le, b_le, W_T1, b_T1, W_T2,
                 ln1_g, ln1_b, W_b1, b_b1, W_b2, b_b2, h_g, h_b, W_h, b_h)
    return out


# speedup vs baseline: 1.6936x; 1.6936x over previous
import functools

import jax
import jax.numpy as jnp
from jax import lax
from jax.experimental import pallas as pl
from jax.experimental.pallas import tpu as pltpu
from jax.experimental.pallas import tpu_sc as plsc

B = 1024
N_CAND = 50000
N_FEAT = 96
D_MAIN = 128
D_BLOCK = 256
CTX = 96

NC_PAD = 50176
N_CHUNK = NC_PAD // 128
CMIN_PAD = 512
AUG = 144
I32_MAX = 2147483647


def _enc_cand_body(x_ref, wi_ref, bi_ref, wk_ref, bk_ref, tbl_ref, aug_ref):
    i = pl.program_id(0)
    cx = jnp.dot(x_ref[:], wi_ref[:], preferred_element_type=jnp.float32) + bi_ref[:]
    ck = jnp.dot(cx, wk_ref[:], preferred_element_type=jnp.float32) + bk_ref[:]
    tbl_ref[:] = ck
    rows = i * 512 + lax.broadcasted_iota(jnp.int32, (512, 1), 0)
    pad = rows >= N_CAND
    cksq = jnp.sum(ck * ck, axis=-1, keepdims=True)
    cksq = jnp.where(pad, 1e30, cksq)
    one = jnp.ones((512, 1), jnp.float32)
    zero = jnp.zeros((512, AUG - 130), jnp.float32)
    aug_ref[:] = jnp.concatenate([ck, cksq, one, zero], axis=1)


def _encode_candidates(cand_p, W_in, b_in, W_K, b_K):
    return pl.pallas_call(
        _enc_cand_body,
        grid=(NC_PAD // 512,),
        in_specs=[
            pl.BlockSpec((512, N_FEAT), lambda i: (i, 0)),
            pl.BlockSpec((N_FEAT, D_MAIN), lambda i: (0, 0)),
            pl.BlockSpec((1, D_MAIN), lambda i: (0, 0)),
            pl.BlockSpec((D_MAIN, D_MAIN), lambda i: (0, 0)),
            pl.BlockSpec((1, D_MAIN), lambda i: (0, 0)),
        ],
        out_specs=[
            pl.BlockSpec((512, D_MAIN), lambda i: (i, 0)),
            pl.BlockSpec((512, AUG), lambda i: (i, 0)),
        ],
        out_shape=[
            jax.ShapeDtypeStruct((NC_PAD, D_MAIN), jnp.float32),
            jax.ShapeDtypeStruct((NC_PAD, AUG), jnp.float32),
        ],
    )(cand_p, W_in, b_in, W_K, b_K)


def _enc_query_body(x_ref, wi_ref, bi_ref, wk_ref, bk_ref, x1_ref, k_ref, kaug_ref):
    x1 = jnp.dot(x_ref[:], wi_ref[:], preferred_element_type=jnp.float32) + bi_ref[:]
    k = jnp.dot(x1, wk_ref[:], preferred_element_type=jnp.float32) + bk_ref[:]
    x1_ref[:] = x1
    k_ref[:] = k
    ksq = jnp.sum(k * k, axis=-1, keepdims=True)
    one = jnp.ones((256, 1), jnp.float32)
    zero = jnp.zeros((256, AUG - 130), jnp.float32)
    kaug_ref[:] = jnp.concatenate([-2.0 * k, one, ksq, zero], axis=1)


def _encode_queries(x_num, W_in, b_in, W_K, b_K):
    return pl.pallas_call(
        _enc_query_body,
        grid=(B // 256,),
        in_specs=[
            pl.BlockSpec((256, N_FEAT), lambda i: (i, 0)),
            pl.BlockSpec((N_FEAT, D_MAIN), lambda i: (0, 0)),
            pl.BlockSpec((1, D_MAIN), lambda i: (0, 0)),
            pl.BlockSpec((D_MAIN, D_MAIN), lambda i: (0, 0)),
            pl.BlockSpec((1, D_MAIN), lambda i: (0, 0)),
        ],
        out_specs=[
            pl.BlockSpec((256, D_MAIN), lambda i: (i, 0)),
            pl.BlockSpec((256, D_MAIN), lambda i: (i, 0)),
            pl.BlockSpec((256, AUG), lambda i: (i, 0)),
        ],
        out_shape=[
            jax.ShapeDtypeStruct((B, D_MAIN), jnp.float32),
            jax.ShapeDtypeStruct((B, D_MAIN), jnp.float32),
            jax.ShapeDtypeStruct((B, AUG), jnp.float32),
        ],
    )(x_num, W_in, b_in, W_K, b_K)


def _dist_body(kaug_ref, caug_ref, keys_ref):
    d2 = lax.dot_general(kaug_ref[:], caug_ref[:], (((1,), (1,)), ((), ())),
                         preferred_element_type=jnp.float32)
    u = lax.bitcast_convert_type(d2, jnp.int32)
    keys_ref[:] = jnp.where(u < 0, u ^ 0x7FFFFFFF, u)


def _distance_keys(k_aug, ck_aug):
    RB, CB = 256, 6272
    return pl.pallas_call(
        _dist_body,
        grid=(B // RB, NC_PAD // CB),
        in_specs=[
            pl.BlockSpec((RB, AUG), lambda i, j: (i, 0)),
            pl.BlockSpec((CB, AUG), lambda i, j: (j, 0)),
        ],
        out_specs=pl.BlockSpec((RB, CB), lambda i, j: (i, j)),
        out_shape=jax.ShapeDtypeStruct((B, NC_PAD), jnp.int32),
    )(k_aug, ck_aug)


def _thresh_body(keys_ref, t_ref):
    keys = keys_ref[:]
    rb = keys.shape[0]
    lo0 = jnp.full((rb, 1), -2147483648, jnp.int32)
    hi0 = jnp.full((rb, 1), I32_MAX, jnp.int32)

    def body(_, carry):
        lo, hi = carry
        mid = (lo & hi) + ((lo ^ hi) >> 1)
        cnt = jnp.sum((keys <= mid).astype(jnp.int32), axis=1, keepdims=True)
        ge = cnt >= CTX
        return jnp.where(ge, lo, mid), jnp.where(ge, mid, hi)

    lo, hi = lax.fori_loop(0, 32, body, (lo0, hi0))
    t_ref[:] = jnp.broadcast_to(hi, (rb, 128))


def _threshold(keys):
    RB = 64
    bh = keys.shape[0]
    return pl.pallas_call(
        _thresh_body,
        grid=(bh // RB,),
        in_specs=[pl.BlockSpec((RB, NC_PAD), lambda i: (i, 0))],
        out_specs=pl.BlockSpec((RB, 128), lambda i: (i, 0)),
        out_shape=jax.ShapeDtypeStruct((bh, 128), jnp.int32),
    )(keys)


NV = NC_PAD // 16


def _sc_body(rows_per_tile, keys_hbm, t_hbm, ck_hbm, y_hbm, ctx_hbm, ysel_hbm,
             keys_v, y_v, gath_v, t_v, lt_v, eq_v, idx_v, ys_v, sem):
    wid = lax.axis_index("s") * 2 + lax.axis_index("c")
    pltpu.sync_copy(y_hbm, y_v)
    iota16 = lax.iota(jnp.int32, 16)

    zeros16i = jnp.zeros((16,), jnp.int32)

    def row_body(r, _):
        row = wid * rows_per_tile + r
        pltpu.sync_copy(keys_hbm.at[row], keys_v)
        pltpu.sync_copy(t_hbm.at[row], t_v)
        tvec = t_v[pl.ds(0, 16)]

        def group_body(g, c_le):
            les = []
            for u in range(8):
                v = keys_v[pl.ds((g * 8 + u) * 16, 16)]
                les.append(v <= tvec)
            m = les[0]
            for u in range(1, 8):
                m = m | les[u]

            def hit(c):
                cc = c
                for u in range(8):
                    le = les[u]
                    ivec = (g * 8 + u) * 16 + iota16
                    pos = cc + plsc.cumsum(le.astype(jnp.int32)) - 1
                    plsc.store_scatter(idx_v, [pos], ivec,
                                       mask=le & (pos < CTX))
                    cc = cc + plsc.all_reduce_population_count(le)
                return cc
            return lax.cond(jnp.any(m), hit, lambda c: c, c_le)

        c_le = lax.fori_loop(0, NV // 8, group_body, zeros16i)

        def slow(_):
            def vreg_body(j, carry):
                cnt_lt, cnt_eq = carry
                v = keys_v[pl.ds(j * 16, 16)]
                le = v <= tvec

                def hit2(c):
                    c_lt, c_eq = c
                    lt = v < tvec
                    eq = le & (~lt)
                    ivec = j * 16 + iota16
                    pos_lt = c_lt + plsc.cumsum(lt.astype(jnp.int32)) - 1
                    pos_eq = c_eq + plsc.cumsum(eq.astype(jnp.int32)) - 1
                    plsc.store_scatter(lt_v, [pos_lt], ivec,
                                       mask=lt & (pos_lt < CTX))
                    plsc.store_scatter(eq_v, [pos_eq], ivec,
                                       mask=eq & (pos_eq < CTX))
                    return (c_lt + plsc.all_reduce_population_count(lt),
                            c_eq + plsc.all_reduce_population_count(eq))

                return lax.cond(jnp.any(le), hit2, lambda c: c,
                                (cnt_lt, cnt_eq))

            clt, _ceq = lax.fori_loop(0, NV, vreg_body,
                                      (zeros16i, zeros16i))
            for jv in range(CTX // 16):
                jvec = jv * 16 + iota16
                a = plsc.load_gather(lt_v, [jnp.minimum(jvec, CTX - 1)])
                b = plsc.load_gather(eq_v, [jnp.clip(jvec - clt, 0, CTX - 1)])
                sel = jnp.where(jvec < clt, a, b)
                idx_v[pl.ds(jv * 16, 16)] = sel
            return 0

        lax.cond(jnp.any(c_le != CTX), slow, lambda _: 0, 0)

        for jv in range(CTX // 16):
            sel = idx_v[pl.ds(jv * 16, 16)]
            ys_v[pl.ds(jv * 16, 16)] = plsc.load_gather(y_v, [sel])
        zeros16 = jnp.zeros((16,), jnp.float32)
        ys_v[pl.ds(96, 16)] = zeros16
        ys_v[pl.ds(112, 16)] = zeros16
        pltpu.async_copy(ck_hbm.at[idx_v], gath_v, sem).wait()
        pltpu.sync_copy(gath_v, ctx_hbm.at[pl.ds(row * CTX, CTX)])
        pltpu.sync_copy(ys_v, ysel_hbm.at[row])
        return 0

    lax.fori_loop(0, rows_per_tile, row_body, 0)


def _sc_select_gather(keys, t_bcast, ck_tbl, y_p):
    bh = keys.shape[0]
    f = functools.partial(
        pl.kernel,
        mesh=plsc.VectorSubcoreMesh(core_axis_name="c", subcore_axis_name="s"),
        compiler_params=pltpu.CompilerParams(needs_layout_passes=False),
        out_type=[
            jax.ShapeDtypeStruct((bh * CTX, D_MAIN), jnp.float32),
            jax.ShapeDtypeStruct((bh, 128), jnp.float32),
        ],
        scratch_types=[
            pltpu.VMEM((NC_PAD,), jnp.int32),
            pltpu.VMEM((NC_PAD,), jnp.float32),
            pltpu.VMEM((CTX, D_MAIN), jnp.float32),
            pltpu.VMEM((128,), jnp.int32),
            pltpu.VMEM((CTX,), jnp.int32),
            pltpu.VMEM((CTX,), jnp.int32),
            pltpu.VMEM((CTX,), jnp.int32),
            pltpu.VMEM((128,), jnp.float32),
            pltpu.SemaphoreType.DMA,
        ],
    )(functools.partial(_sc_body, bh // 32))
    return f(keys, t_bcast, ck_tbl, y_p)


def _final_body(x1_ref, k_ref, ctx_ref, ys_ref,
                wle_ref, ble_ref, wt1_ref, bt1_ref, wt2_ref,
                g1_ref, bb1_ref, wb1_ref, b_b1_ref, wb2_ref, b_b2_ref,
                hg_ref, hb_ref, wh_ref, bh_ref, out_ref):
    rb = x1_ref.shape[0]
    k = k_ref[:]
    ksq = jnp.sum(k * k, axis=-1, keepdims=True)
    ck3 = ctx_ref[:].reshape(rb, CTX, D_MAIN)
    kb = k[:, None, :]
    sims = (-ksq + 2.0 * jnp.sum(kb * ck3, axis=-1)
            - jnp.sum(ck3 * ck3, axis=-1))
    m = jnp.max(sims, axis=-1, keepdims=True)
    e = jnp.exp(sims - m)
    probs = e / jnp.sum(e, axis=-1, keepdims=True)

    t_in = (kb - ck3).reshape(rb * CTX, D_MAIN)
    hid = jnp.maximum(
        jnp.dot(t_in, wt1_ref[:], preferred_element_type=jnp.float32) + bt1_ref[:], 0.0)
    v = jnp.dot(hid, wt2_ref[:], preferred_element_type=jnp.float32)
    w3 = wle_ref[:].reshape(1, 1, D_MAIN)
    b3 = ble_ref[:].reshape(1, 1, D_MAIN)
    v3 = v.reshape(rb, CTX, D_MAIN) + ys_ref[:][:, :, None] * w3 + b3
    ctxx = jnp.sum(probs[:, :, None] * v3, axis=1)
    x = x1_ref[:] + ctxx

    mu = x.mean(-1, keepdims=True)
    var = ((x - mu) ** 2).mean(-1, keepdims=True)
    h = (x - mu) / jnp.sqrt(var + 1e-5) * g1_ref[:] + bb1_ref[:]
    x = x + (jnp.maximum(
        jnp.dot(h, wb1_ref[:], preferred_element_type=jnp.float32) + b_b1_ref[:], 0.0)
        @ wb2_ref[:] + b_b2_ref[:])
    mu = x.mean(-1, keepdims=True)
    var = ((x - mu) ** 2).mean(-1, keepdims=True)
    h = (x - mu) / jnp.sqrt(var + 1e-5) * hg_ref[:] + hb_ref[:]
    out = jnp.dot(jnp.maximum(h, 0.0), wh_ref[:],
                  preferred_element_type=jnp.float32) + bh_ref[:]
    out_ref[:] = out


def _final(x1, k, ctx_rows, ysel, W_le, b_le, W_T1, b_T1, W_T2,
           ln1_g, ln1_b, W_b1, b_b1, W_b2, b_b2, h_g, h_b, W_h, b_h):
    RB = 128
    bh = x1.shape[0]
    r2 = lambda a: a.reshape(1, -1)
    return pl.pallas_call(
        _final_body,
        grid=(bh // RB,),
        in_specs=[
            pl.BlockSpec((RB, D_MAIN), lambda i: (i, 0)),
            pl.BlockSpec((RB, D_MAIN), lambda i: (i, 0)),
            pl.BlockSpec((RB * CTX, D_MAIN), lambda i: (i, 0)),
            pl.BlockSpec((RB, CTX), lambda i: (i, 0)),
            pl.BlockSpec((1, D_MAIN), lambda i: (0, 0)),
            pl.BlockSpec((1, D_MAIN), lambda i: (0, 0)),
            pl.BlockSpec((D_MAIN, D_BLOCK), lambda i: (0, 0)),
            pl.BlockSpec((1, D_BLOCK), lambda i: (0, 0)),
            pl.BlockSpec((D_BLOCK, D_MAIN), lambda i: (0, 0)),
            pl.BlockSpec((1, D_MAIN), lambda i: (0, 0)),
            pl.BlockSpec((1, D_MAIN), lambda i: (0, 0)),
            pl.BlockSpec((D_MAIN, D_BLOCK), lambda i: (0, 0)),
            pl.BlockSpec((1, D_BLOCK), lambda i: (0, 0)),
            pl.BlockSpec((D_BLOCK, D_MAIN), lambda i: (0, 0)),
            pl.BlockSpec((1, D_MAIN), lambda i: (0, 0)),
            pl.BlockSpec((1, D_MAIN), lambda i: (0, 0)),
            pl.BlockSpec((1, D_MAIN), lambda i: (0, 0)),
            pl.BlockSpec((D_MAIN, 1), lambda i: (0, 0)),
            pl.BlockSpec((1, 1), lambda i: (0, 0)),
        ],
        out_specs=pl.BlockSpec((RB, 1), lambda i: (i, 0)),
        out_shape=jax.ShapeDtypeStruct((bh, 1), jnp.float32),
    )(x1, k, ctx_rows, ysel, r2(W_le), r2(b_le), W_T1, r2(b_T1), W_T2,
      r2(ln1_g), r2(ln1_b), W_b1, r2(b_b1), W_b2, r2(b_b2), r2(h_g), r2(h_b),
      W_h, b_h.reshape(1, 1))


def kernel(x_num, candidate_x_num, candidate_y, W_in, b_in, W_K, b_K, W_le, b_le,
           W_T1, b_T1, W_T2, ln1_g, ln1_b, W_b1, b_b1, W_b2, b_b2, h_g, h_b, W_h, b_h,
           context_size):
    cand_p = jnp.pad(candidate_x_num, ((0, NC_PAD - N_CAND), (0, 0)))
    y_p = jnp.pad(candidate_y, (0, NC_PAD - N_CAND))
    bi = b_in.reshape(1, -1)
    bk = b_K.reshape(1, -1)

    ck_tbl, ck_aug = _encode_candidates(cand_p, W_in, bi, W_K, bk)
    x1, k, k_aug = _encode_queries(x_num, W_in, bi, W_K, bk)
    keys = _distance_keys(k_aug, ck_aug)

    outs = []
    H = B // 2
    for h in range(2):
        sl = slice(h * H, (h + 1) * H)
        keys_h = lax.slice(keys, (h * H, 0), ((h + 1) * H, NC_PAD))
        t_h = _threshold(keys_h)
        ctx_h, ysel_pad_h = _sc_select_gather(keys_h, t_h, ck_tbl, y_p)
        outs.append(_final(x1[sl], k[sl], ctx_h, ysel_pad_h[:, :CTX],
                           W_le, b_le, W_T1, b_T1, W_T2,
                           ln1_g, ln1_b, W_b1, b_b1, W_b2, b_b2,
                           h_g, h_b, W_h, b_h))
    return jnp.concatenate(outs, axis=0)


# speedup vs baseline: 1.8396x; 1.0862x over previous
import functools

import jax
import jax.numpy as jnp
from jax import lax
from jax.experimental import pallas as pl
from jax.experimental.pallas import tpu as pltpu
from jax.experimental.pallas import tpu_sc as plsc

B = 1024
N_CAND = 50000
N_FEAT = 96
D_MAIN = 128
D_BLOCK = 256
CTX = 96

NC_PAD = 50176
N_CHUNK = NC_PAD // 128
CMIN_PAD = 512
AUG = 144
I32_MAX = 2147483647


def _enc_cand_body(x_ref, wi_ref, bi_ref, wk_ref, bk_ref, tbl_ref, aug_ref):
    i = pl.program_id(0)
    cx = jnp.dot(x_ref[:], wi_ref[:], preferred_element_type=jnp.float32) + bi_ref[:]
    ck = jnp.dot(cx, wk_ref[:], preferred_element_type=jnp.float32) + bk_ref[:]
    tbl_ref[:] = ck
    rows = i * 512 + lax.broadcasted_iota(jnp.int32, (512, 1), 0)
    pad = rows >= N_CAND
    cksq = jnp.sum(ck * ck, axis=-1, keepdims=True)
    cksq = jnp.where(pad, 1e30, cksq)
    one = jnp.ones((512, 1), jnp.float32)
    zero = jnp.zeros((512, AUG - 130), jnp.float32)
    aug_ref[:] = jnp.concatenate([ck, cksq, one, zero], axis=1)


def _encode_candidates(cand_p, W_in, b_in, W_K, b_K):
    return pl.pallas_call(
        _enc_cand_body,
        grid=(NC_PAD // 512,),
        in_specs=[
            pl.BlockSpec((512, N_FEAT), lambda i: (i, 0)),
            pl.BlockSpec((N_FEAT, D_MAIN), lambda i: (0, 0)),
            pl.BlockSpec((1, D_MAIN), lambda i: (0, 0)),
            pl.BlockSpec((D_MAIN, D_MAIN), lambda i: (0, 0)),
            pl.BlockSpec((1, D_MAIN), lambda i: (0, 0)),
        ],
        out_specs=[
            pl.BlockSpec((512, D_MAIN), lambda i: (i, 0)),
            pl.BlockSpec((512, AUG), lambda i: (i, 0)),
        ],
        out_shape=[
            jax.ShapeDtypeStruct((NC_PAD, D_MAIN), jnp.float32),
            jax.ShapeDtypeStruct((NC_PAD, AUG), jnp.float32),
        ],
    )(cand_p, W_in, b_in, W_K, b_K)


def _enc_query_body(x_ref, wi_ref, bi_ref, wk_ref, bk_ref, x1_ref, k_ref, kaug_ref):
    x1 = jnp.dot(x_ref[:], wi_ref[:], preferred_element_type=jnp.float32) + bi_ref[:]
    k = jnp.dot(x1, wk_ref[:], preferred_element_type=jnp.float32) + bk_ref[:]
    x1_ref[:] = x1
    k_ref[:] = k
    ksq = jnp.sum(k * k, axis=-1, keepdims=True)
    one = jnp.ones((256, 1), jnp.float32)
    zero = jnp.zeros((256, AUG - 130), jnp.float32)
    kaug_ref[:] = jnp.concatenate([-2.0 * k, one, ksq, zero], axis=1)


def _encode_queries(x_num, W_in, b_in, W_K, b_K):
    return pl.pallas_call(
        _enc_query_body,
        grid=(B // 256,),
        in_specs=[
            pl.BlockSpec((256, N_FEAT), lambda i: (i, 0)),
            pl.BlockSpec((N_FEAT, D_MAIN), lambda i: (0, 0)),
            pl.BlockSpec((1, D_MAIN), lambda i: (0, 0)),
            pl.BlockSpec((D_MAIN, D_MAIN), lambda i: (0, 0)),
            pl.BlockSpec((1, D_MAIN), lambda i: (0, 0)),
        ],
        out_specs=[
            pl.BlockSpec((256, D_MAIN), lambda i: (i, 0)),
            pl.BlockSpec((256, D_MAIN), lambda i: (i, 0)),
            pl.BlockSpec((256, AUG), lambda i: (i, 0)),
        ],
        out_shape=[
            jax.ShapeDtypeStruct((B, D_MAIN), jnp.float32),
            jax.ShapeDtypeStruct((B, D_MAIN), jnp.float32),
            jax.ShapeDtypeStruct((B, AUG), jnp.float32),
        ],
    )(x_num, W_in, b_in, W_K, b_K)


def _dist_body(kaug_ref, caug_ref, keys_ref):
    d2 = lax.dot_general(kaug_ref[:], caug_ref[:], (((1,), (1,)), ((), ())),
                         preferred_element_type=jnp.float32)
    u = lax.bitcast_convert_type(d2, jnp.int32)
    keys_ref[:] = jnp.where(u < 0, u ^ 0x7FFFFFFF, u)


def _distance_keys(k_aug, ck_aug):
    RB, CB = 256, 6272
    return pl.pallas_call(
        _dist_body,
        grid=(B // RB, NC_PAD // CB),
        in_specs=[
            pl.BlockSpec((RB, AUG), lambda i, j: (i, 0)),
            pl.BlockSpec((CB, AUG), lambda i, j: (j, 0)),
        ],
        out_specs=pl.BlockSpec((RB, CB), lambda i, j: (i, j)),
        out_shape=jax.ShapeDtypeStruct((B, NC_PAD), jnp.int32),
    )(k_aug, ck_aug)


def _thresh_body(keys_ref, t_ref):
    keys = keys_ref[:]
    rb = keys.shape[0]
    lo0 = jnp.full((rb, 1), -2147483648, jnp.int32)
    hi0 = jnp.full((rb, 1), I32_MAX, jnp.int32)

    def body(_, carry):
        lo, hi = carry
        mid = (lo & hi) + ((lo ^ hi) >> 1)
        cnt = jnp.sum((keys <= mid).astype(jnp.int32), axis=1, keepdims=True)
        ge = cnt >= CTX
        return jnp.where(ge, lo, mid), jnp.where(ge, mid, hi)

    lo, hi = lax.fori_loop(0, 32, body, (lo0, hi0))
    t_ref[:] = jnp.broadcast_to(hi, (rb, 128))


def _threshold(keys):
    RB = 64
    bh = keys.shape[0]
    return pl.pallas_call(
        _thresh_body,
        grid=(bh // RB,),
        in_specs=[pl.BlockSpec((RB, NC_PAD), lambda i: (i, 0))],
        out_specs=pl.BlockSpec((RB, 128), lambda i: (i, 0)),
        out_shape=jax.ShapeDtypeStruct((bh, 128), jnp.int32),
    )(keys)


NV = NC_PAD // 16


def _sc_body(rows_per_tile, keys_hbm, t_hbm, ck_hbm, y_hbm, ctx_hbm, ysel_hbm,
             keys_v, y_v, gath_v, t_v, lt_v, eq_v, idx_v, ys_v, sem):
    wid = lax.axis_index("s") * 2 + lax.axis_index("c")
    pltpu.sync_copy(y_hbm, y_v)
    iota16 = lax.iota(jnp.int32, 16)

    zeros16i = jnp.zeros((16,), jnp.int32)

    def row_body(r, _):
        row = wid * rows_per_tile + r
        pltpu.sync_copy(keys_hbm.at[row], keys_v)
        pltpu.sync_copy(t_hbm.at[row], t_v)
        tvec = t_v[pl.ds(0, 16)]

        def group_body(g, c_le):
            les = []
            for u in range(8):
                v = keys_v[pl.ds((g * 8 + u) * 16, 16)]
                les.append(v <= tvec)
            m = les[0]
            for u in range(1, 8):
                m = m | les[u]

            def hit(c):
                cc = c
                for u in range(8):
                    le = les[u]
                    ivec = (g * 8 + u) * 16 + iota16
                    pos = cc + plsc.cumsum(le.astype(jnp.int32)) - 1
                    plsc.store_scatter(idx_v, [pos], ivec,
                                       mask=le & (pos < CTX))
                    cc = cc + plsc.all_reduce_population_count(le)
                return cc
            return lax.cond(jnp.any(m), hit, lambda c: c, c_le)

        c_le = lax.fori_loop(0, NV // 8, group_body, zeros16i)

        def slow(_):
            def vreg_body(j, carry):
                cnt_lt, cnt_eq = carry
                v = keys_v[pl.ds(j * 16, 16)]
                le = v <= tvec

                def hit2(c):
                    c_lt, c_eq = c
                    lt = v < tvec
                    eq = le & (~lt)
                    ivec = j * 16 + iota16
                    pos_lt = c_lt + plsc.cumsum(lt.astype(jnp.int32)) - 1
                    pos_eq = c_eq + plsc.cumsum(eq.astype(jnp.int32)) - 1
                    plsc.store_scatter(lt_v, [pos_lt], ivec,
                                       mask=lt & (pos_lt < CTX))
                    plsc.store_scatter(eq_v, [pos_eq], ivec,
                                       mask=eq & (pos_eq < CTX))
                    return (c_lt + plsc.all_reduce_population_count(lt),
                            c_eq + plsc.all_reduce_population_count(eq))

                return lax.cond(jnp.any(le), hit2, lambda c: c,
                                (cnt_lt, cnt_eq))

            clt, _ceq = lax.fori_loop(0, NV, vreg_body,
                                      (zeros16i, zeros16i))
            for jv in range(CTX // 16):
                jvec = jv * 16 + iota16
                a = plsc.load_gather(lt_v, [jnp.minimum(jvec, CTX - 1)])
                b = plsc.load_gather(eq_v, [jnp.clip(jvec - clt, 0, CTX - 1)])
                sel = jnp.where(jvec < clt, a, b)
                idx_v[pl.ds(jv * 16, 16)] = sel
            return 0

        lax.cond(jnp.any(c_le != CTX), slow, lambda _: 0, 0)

        for jv in range(CTX // 16):
            sel = idx_v[pl.ds(jv * 16, 16)]
            ys_v[pl.ds(jv * 16, 16)] = plsc.load_gather(y_v, [sel])
        zeros16 = jnp.zeros((16,), jnp.float32)
        ys_v[pl.ds(96, 16)] = zeros16
        ys_v[pl.ds(112, 16)] = zeros16
        pltpu.async_copy(ck_hbm.at[idx_v], gath_v, sem).wait()
        pltpu.sync_copy(gath_v, ctx_hbm.at[pl.ds(row * CTX, CTX)])
        pltpu.sync_copy(ys_v, ysel_hbm.at[row])
        return 0

    lax.fori_loop(0, rows_per_tile, row_body, 0)


def _sc_select_gather(keys, t_bcast, ck_tbl, y_p):
    bh = keys.shape[0]
    f = functools.partial(
        pl.kernel,
        mesh=plsc.VectorSubcoreMesh(core_axis_name="c", subcore_axis_name="s"),
        compiler_params=pltpu.CompilerParams(needs_layout_passes=False),
        out_type=[
            jax.ShapeDtypeStruct((bh * CTX, D_MAIN), jnp.float32),
            jax.ShapeDtypeStruct((bh, 128), jnp.float32),
        ],
        scratch_types=[
            pltpu.VMEM((NC_PAD,), jnp.int32),
            pltpu.VMEM((NC_PAD,), jnp.float32),
            pltpu.VMEM((CTX, D_MAIN), jnp.float32),
            pltpu.VMEM((128,), jnp.int32),
            pltpu.VMEM((CTX,), jnp.int32),
            pltpu.VMEM((CTX,), jnp.int32),
            pltpu.VMEM((CTX,), jnp.int32),
            pltpu.VMEM((128,), jnp.float32),
            pltpu.SemaphoreType.DMA,
        ],
    )(functools.partial(_sc_body, bh // 32))
    return f(keys, t_bcast, ck_tbl, y_p)


def _final_body(x1_ref, k_ref, ctx_ref, ys_ref,
                wle_ref, ble_ref, wt1_ref, bt1_ref, wt2_ref,
                g1_ref, bb1_ref, wb1_ref, b_b1_ref, wb2_ref, b_b2_ref,
                hg_ref, hb_ref, wh_ref, bh_ref, out_ref):
    rb = x1_ref.shape[0]
    k = k_ref[:]
    ksq = jnp.sum(k * k, axis=-1, keepdims=True)
    ck3 = ctx_ref[:].reshape(rb, CTX, D_MAIN)
    kb = k[:, None, :]
    sims = (-ksq + 2.0 * jnp.sum(kb * ck3, axis=-1)
            - jnp.sum(ck3 * ck3, axis=-1))
    m = jnp.max(sims, axis=-1, keepdims=True)
    e = jnp.exp(sims - m)
    probs = e / jnp.sum(e, axis=-1, keepdims=True)

    t_in = (kb - ck3).reshape(rb * CTX, D_MAIN)
    hid = jnp.maximum(
        jnp.dot(t_in, wt1_ref[:], preferred_element_type=jnp.float32) + bt1_ref[:], 0.0)
    v = jnp.dot(hid, wt2_ref[:], preferred_element_type=jnp.float32)
    w3 = wle_ref[:].reshape(1, 1, D_MAIN)
    b3 = ble_ref[:].reshape(1, 1, D_MAIN)
    v3 = v.reshape(rb, CTX, D_MAIN) + ys_ref[:][:, :, None] * w3 + b3
    ctxx = jnp.sum(probs[:, :, None] * v3, axis=1)
    x = x1_ref[:] + ctxx

    mu = x.mean(-1, keepdims=True)
    var = ((x - mu) ** 2).mean(-1, keepdims=True)
    h = (x - mu) / jnp.sqrt(var + 1e-5) * g1_ref[:] + bb1_ref[:]
    x = x + (jnp.maximum(
        jnp.dot(h, wb1_ref[:], preferred_element_type=jnp.float32) + b_b1_ref[:], 0.0)
        @ wb2_ref[:] + b_b2_ref[:])
    mu = x.mean(-1, keepdims=True)
    var = ((x - mu) ** 2).mean(-1, keepdims=True)
    h = (x - mu) / jnp.sqrt(var + 1e-5) * hg_ref[:] + hb_ref[:]
    out = jnp.dot(jnp.maximum(h, 0.0), wh_ref[:],
                  preferred_element_type=jnp.float32) + bh_ref[:]
    out_ref[:] = out


def _final(x1, k, ctx_rows, ysel, W_le, b_le, W_T1, b_T1, W_T2,
           ln1_g, ln1_b, W_b1, b_b1, W_b2, b_b2, h_g, h_b, W_h, b_h):
    RB = 128
    bh = x1.shape[0]
    r2 = lambda a: a.reshape(1, -1)
    return pl.pallas_call(
        _final_body,
        grid=(bh // RB,),
        in_specs=[
            pl.BlockSpec((RB, D_MAIN), lambda i: (i, 0)),
            pl.BlockSpec((RB, D_MAIN), lambda i: (i, 0)),
            pl.BlockSpec((RB * CTX, D_MAIN), lambda i: (i, 0)),
            pl.BlockSpec((RB, CTX), lambda i: (i, 0)),
            pl.BlockSpec((1, D_MAIN), lambda i: (0, 0)),
            pl.BlockSpec((1, D_MAIN), lambda i: (0, 0)),
            pl.BlockSpec((D_MAIN, D_BLOCK), lambda i: (0, 0)),
            pl.BlockSpec((1, D_BLOCK), lambda i: (0, 0)),
            pl.BlockSpec((D_BLOCK, D_MAIN), lambda i: (0, 0)),
            pl.BlockSpec((1, D_MAIN), lambda i: (0, 0)),
            pl.BlockSpec((1, D_MAIN), lambda i: (0, 0)),
            pl.BlockSpec((D_MAIN, D_BLOCK), lambda i: (0, 0)),
            pl.BlockSpec((1, D_BLOCK), lambda i: (0, 0)),
            pl.BlockSpec((D_BLOCK, D_MAIN), lambda i: (0, 0)),
            pl.BlockSpec((1, D_MAIN), lambda i: (0, 0)),
            pl.BlockSpec((1, D_MAIN), lambda i: (0, 0)),
            pl.BlockSpec((1, D_MAIN), lambda i: (0, 0)),
            pl.BlockSpec((D_MAIN, 1), lambda i: (0, 0)),
            pl.BlockSpec((1, 1), lambda i: (0, 0)),
        ],
        out_specs=pl.BlockSpec((RB, 1), lambda i: (i, 0)),
        out_shape=jax.ShapeDtypeStruct((bh, 1), jnp.float32),
    )(x1, k, ctx_rows, ysel, r2(W_le), r2(b_le), W_T1, r2(b_T1), W_T2,
      r2(ln1_g), r2(ln1_b), W_b1, r2(b_b1), W_b2, r2(b_b2), r2(h_g), r2(h_b),
      W_h, b_h.reshape(1, 1))


def kernel(x_num, candidate_x_num, candidate_y, W_in, b_in, W_K, b_K, W_le, b_le,
           W_T1, b_T1, W_T2, ln1_g, ln1_b, W_b1, b_b1, W_b2, b_b2, h_g, h_b, W_h, b_h,
           context_size):
    cand_p = jnp.pad(candidate_x_num, ((0, NC_PAD - N_CAND), (0, 0)))
    y_p = jnp.pad(candidate_y, (0, NC_PAD - N_CAND))
    bi = b_in.reshape(1, -1)
    bk = b_K.reshape(1, -1)

    ck_tbl, ck_aug = _encode_candidates(cand_p, W_in, bi, W_K, bk)
    x1, k, k_aug = _encode_queries(x_num, W_in, bi, W_K, bk)
    keys = _distance_keys(k_aug, ck_aug)

    outs = []
    H = B // 4
    for h in range(4):
        sl = slice(h * H, (h + 1) * H)
        keys_h = lax.slice(keys, (h * H, 0), ((h + 1) * H, NC_PAD))
        t_h = _threshold(keys_h)
        ctx_h, ysel_pad_h = _sc_select_gather(keys_h, t_h, ck_tbl, y_p)
        outs.append(_final(x1[sl], k[sl], ctx_h, ysel_pad_h[:, :CTX],
                           W_le, b_le, W_T1, b_T1, W_T2,
                           ln1_g, ln1_b, W_b1, b_b1, W_b2, b_b2,
                           h_g, h_b, W_h, b_h))
    return jnp.concatenate(outs, axis=0)


# speedup vs baseline: 1.8735x; 1.0184x over previous
import functools

import jax
import jax.numpy as jnp
from jax import lax
from jax.experimental import pallas as pl
from jax.experimental.pallas import tpu as pltpu
from jax.experimental.pallas import tpu_sc as plsc

B = 1024
N_CAND = 50000
N_FEAT = 96
D_MAIN = 128
D_BLOCK = 256
CTX = 96

NC_PAD = 50176
N_CHUNK = NC_PAD // 128
CMIN_PAD = 512
AUG = 144
I32_MAX = 2147483647


def _enc_cand_body(x_ref, wi_ref, bi_ref, wk_ref, bk_ref, tbl_ref, aug_ref):
    i = pl.program_id(0)
    cx = jnp.dot(x_ref[:], wi_ref[:], preferred_element_type=jnp.float32) + bi_ref[:]
    ck = jnp.dot(cx, wk_ref[:], preferred_element_type=jnp.float32) + bk_ref[:]
    tbl_ref[:] = ck
    rows = i * 512 + lax.broadcasted_iota(jnp.int32, (512, 1), 0)
    pad = rows >= N_CAND
    cksq = jnp.sum(ck * ck, axis=-1, keepdims=True)
    cksq = jnp.where(pad, 1e30, cksq)
    one = jnp.ones((512, 1), jnp.float32)
    zero = jnp.zeros((512, AUG - 130), jnp.float32)
    aug_ref[:] = jnp.concatenate([ck, cksq, one, zero], axis=1)


def _encode_candidates(cand_p, W_in, b_in, W_K, b_K):
    return pl.pallas_call(
        _enc_cand_body,
        grid=(NC_PAD // 512,),
        in_specs=[
            pl.BlockSpec((512, N_FEAT), lambda i: (i, 0)),
            pl.BlockSpec((N_FEAT, D_MAIN), lambda i: (0, 0)),
            pl.BlockSpec((1, D_MAIN), lambda i: (0, 0)),
            pl.BlockSpec((D_MAIN, D_MAIN), lambda i: (0, 0)),
            pl.BlockSpec((1, D_MAIN), lambda i: (0, 0)),
        ],
        out_specs=[
            pl.BlockSpec((512, D_MAIN), lambda i: (i, 0)),
            pl.BlockSpec((512, AUG), lambda i: (i, 0)),
        ],
        out_shape=[
            jax.ShapeDtypeStruct((NC_PAD, D_MAIN), jnp.float32),
            jax.ShapeDtypeStruct((NC_PAD, AUG), jnp.float32),
        ],
    )(cand_p, W_in, b_in, W_K, b_K)


def _enc_query_body(x_ref, wi_ref, bi_ref, wk_ref, bk_ref, x1_ref, k_ref, kaug_ref):
    x1 = jnp.dot(x_ref[:], wi_ref[:], preferred_element_type=jnp.float32) + bi_ref[:]
    k = jnp.dot(x1, wk_ref[:], preferred_element_type=jnp.float32) + bk_ref[:]
    x1_ref[:] = x1
    k_ref[:] = k
    ksq = jnp.sum(k * k, axis=-1, keepdims=True)
    one = jnp.ones((256, 1), jnp.float32)
    zero = jnp.zeros((256, AUG - 130), jnp.float32)
    kaug_ref[:] = jnp.concatenate([-2.0 * k, one, ksq, zero], axis=1)


def _encode_queries(x_num, W_in, b_in, W_K, b_K):
    return pl.pallas_call(
        _enc_query_body,
        grid=(B // 256,),
        in_specs=[
            pl.BlockSpec((256, N_FEAT), lambda i: (i, 0)),
            pl.BlockSpec((N_FEAT, D_MAIN), lambda i: (0, 0)),
            pl.BlockSpec((1, D_MAIN), lambda i: (0, 0)),
            pl.BlockSpec((D_MAIN, D_MAIN), lambda i: (0, 0)),
            pl.BlockSpec((1, D_MAIN), lambda i: (0, 0)),
        ],
        out_specs=[
            pl.BlockSpec((256, D_MAIN), lambda i: (i, 0)),
            pl.BlockSpec((256, D_MAIN), lambda i: (i, 0)),
            pl.BlockSpec((256, AUG), lambda i: (i, 0)),
        ],
        out_shape=[
            jax.ShapeDtypeStruct((B, D_MAIN), jnp.float32),
            jax.ShapeDtypeStruct((B, D_MAIN), jnp.float32),
            jax.ShapeDtypeStruct((B, AUG), jnp.float32),
        ],
    )(x_num, W_in, b_in, W_K, b_K)


def _dist_body(kaug_ref, caug_ref, keys_ref):
    d2 = lax.dot_general(kaug_ref[:], caug_ref[:], (((1,), (1,)), ((), ())),
                         preferred_element_type=jnp.float32)
    u = lax.bitcast_convert_type(d2, jnp.int32)
    keys_ref[:] = jnp.where(u < 0, u ^ 0x7FFFFFFF, u)


def _distance_keys(k_aug, ck_aug):
    RB, CB = 256, 6272
    return pl.pallas_call(
        _dist_body,
        grid=(B // RB, NC_PAD // CB),
        in_specs=[
            pl.BlockSpec((RB, AUG), lambda i, j: (i, 0)),
            pl.BlockSpec((CB, AUG), lambda i, j: (j, 0)),
        ],
        out_specs=pl.BlockSpec((RB, CB), lambda i, j: (i, j)),
        out_shape=jax.ShapeDtypeStruct((B, NC_PAD), jnp.int32),
    )(k_aug, ck_aug)


def _thresh_body(keys_ref, t_ref):
    keys = keys_ref[:]
    rb = keys.shape[0]
    lo0 = jnp.full((rb, 1), -2147483648, jnp.int32)
    hi0 = jnp.full((rb, 1), I32_MAX, jnp.int32)

    def body(_, carry):
        lo, hi = carry
        mid = (lo & hi) + ((lo ^ hi) >> 1)
        cnt = jnp.sum((keys <= mid).astype(jnp.int32), axis=1, keepdims=True)
        ge = cnt >= CTX
        return jnp.where(ge, lo, mid), jnp.where(ge, mid, hi)

    lo, hi = lax.fori_loop(0, 32, body, (lo0, hi0))
    t_ref[:] = jnp.broadcast_to(hi, (rb, 128))


def _threshold(keys):
    RB = 64
    bh = keys.shape[0]
    return pl.pallas_call(
        _thresh_body,
        grid=(bh // RB,),
        in_specs=[pl.BlockSpec((RB, NC_PAD), lambda i: (i, 0))],
        out_specs=pl.BlockSpec((RB, 128), lambda i: (i, 0)),
        out_shape=jax.ShapeDtypeStruct((bh, 128), jnp.int32),
    )(keys)


NV = NC_PAD // 16


def _sc_body(rows_per_tile, keys_hbm, t_hbm, ck_hbm, y_hbm, ctx_hbm, ysel_hbm,
             keys_v, y_v, gath_v, t_v, lt_v, eq_v, idx_v, ys_v, sem):
    wid = lax.axis_index("s") * 2 + lax.axis_index("c")
    pltpu.sync_copy(y_hbm, y_v)
    iota16 = lax.iota(jnp.int32, 16)

    zeros16i = jnp.zeros((16,), jnp.int32)

    def row_body(r, _):
        row = wid * rows_per_tile + r
        pltpu.sync_copy(keys_hbm.at[row], keys_v)
        pltpu.sync_copy(t_hbm.at[row], t_v)
        tvec = t_v[pl.ds(0, 16)]

        def group_body(g, c_le):
            les = []
            for u in range(8):
                v = keys_v[pl.ds((g * 8 + u) * 16, 16)]
                les.append(v <= tvec)
            m = les[0]
            for u in range(1, 8):
                m = m | les[u]

            def hit(c):
                cc = c
                for u in range(8):
                    le = les[u]
                    ivec = (g * 8 + u) * 16 + iota16
                    pos = cc + plsc.cumsum(le.astype(jnp.int32)) - 1
                    plsc.store_scatter(idx_v, [pos], ivec,
                                       mask=le & (pos < CTX))
                    cc = cc + plsc.all_reduce_population_count(le)
                return cc
            return lax.cond(jnp.any(m), hit, lambda c: c, c_le)

        c_le = lax.fori_loop(0, NV // 8, group_body, zeros16i)

        def slow(_):
            def vreg_body(j, carry):
                cnt_lt, cnt_eq = carry
                v = keys_v[pl.ds(j * 16, 16)]
                le = v <= tvec

                def hit2(c):
                    c_lt, c_eq = c
                    lt = v < tvec
                    eq = le & (~lt)
                    ivec = j * 16 + iota16
                    pos_lt = c_lt + plsc.cumsum(lt.astype(jnp.int32)) - 1
                    pos_eq = c_eq + plsc.cumsum(eq.astype(jnp.int32)) - 1
                    plsc.store_scatter(lt_v, [pos_lt], ivec,
                                       mask=lt & (pos_lt < CTX))
                    plsc.store_scatter(eq_v, [pos_eq], ivec,
                                       mask=eq & (pos_eq < CTX))
                    return (c_lt + plsc.all_reduce_population_count(lt),
                            c_eq + plsc.all_reduce_population_count(eq))

                return lax.cond(jnp.any(le), hit2, lambda c: c,
                                (cnt_lt, cnt_eq))

            clt, _ceq = lax.fori_loop(0, NV, vreg_body,
                                      (zeros16i, zeros16i))
            for jv in range(CTX // 16):
                jvec = jv * 16 + iota16
                a = plsc.load_gather(lt_v, [jnp.minimum(jvec, CTX - 1)])
                b = plsc.load_gather(eq_v, [jnp.clip(jvec - clt, 0, CTX - 1)])
                sel = jnp.where(jvec < clt, a, b)
                idx_v[pl.ds(jv * 16, 16)] = sel
            return 0

        lax.cond(jnp.any(c_le != CTX), slow, lambda _: 0, 0)

        for jv in range(CTX // 16):
            sel = idx_v[pl.ds(jv * 16, 16)]
            ys_v[pl.ds(jv * 16, 16)] = plsc.load_gather(y_v, [sel])
        zeros16 = jnp.zeros((16,), jnp.float32)
        ys_v[pl.ds(96, 16)] = zeros16
        ys_v[pl.ds(112, 16)] = zeros16
        pltpu.async_copy(ck_hbm.at[idx_v], gath_v, sem).wait()
        pltpu.sync_copy(gath_v, ctx_hbm.at[pl.ds(row * CTX, CTX)])
        pltpu.sync_copy(ys_v, ysel_hbm.at[row])
        return 0

    lax.fori_loop(0, rows_per_tile, row_body, 0)


def _sc_select_gather(keys, t_bcast, ck_tbl, y_p):
    bh = keys.shape[0]
    f = functools.partial(
        pl.kernel,
        mesh=plsc.VectorSubcoreMesh(core_axis_name="c", subcore_axis_name="s"),
        compiler_params=pltpu.CompilerParams(needs_layout_passes=False),
        out_type=[
            jax.ShapeDtypeStruct((bh * CTX, D_MAIN), jnp.float32),
            jax.ShapeDtypeStruct((bh, 128), jnp.float32),
        ],
        scratch_types=[
            pltpu.VMEM((NC_PAD,), jnp.int32),
            pltpu.VMEM((NC_PAD,), jnp.float32),
            pltpu.VMEM((CTX, D_MAIN), jnp.float32),
            pltpu.VMEM((128,), jnp.int32),
            pltpu.VMEM((CTX,), jnp.int32),
            pltpu.VMEM((CTX,), jnp.int32),
            pltpu.VMEM((CTX,), jnp.int32),
            pltpu.VMEM((128,), jnp.float32),
            pltpu.SemaphoreType.DMA,
        ],
    )(functools.partial(_sc_body, bh // 32))
    return f(keys, t_bcast, ck_tbl, y_p)


def _final_body(x1_ref, k_ref, ctx_ref, ys_ref,
                wle_ref, ble_ref, wt1_ref, bt1_ref, wt2_ref,
                g1_ref, bb1_ref, wb1_ref, b_b1_ref, wb2_ref, b_b2_ref,
                hg_ref, hb_ref, wh_ref, bh_ref, out_ref):
    rb = x1_ref.shape[0]
    k = k_ref[:]
    ksq = jnp.sum(k * k, axis=-1, keepdims=True)
    ck3 = ctx_ref[:].reshape(rb, CTX, D_MAIN)
    kb = k[:, None, :]
    sims = (-ksq + 2.0 * jnp.sum(kb * ck3, axis=-1)
            - jnp.sum(ck3 * ck3, axis=-1))
    m = jnp.max(sims, axis=-1, keepdims=True)
    e = jnp.exp(sims - m)
    probs = e / jnp.sum(e, axis=-1, keepdims=True)

    t_in = (kb - ck3).reshape(rb * CTX, D_MAIN)
    hid = jnp.maximum(
        jnp.dot(t_in, wt1_ref[:], preferred_element_type=jnp.float32) + bt1_ref[:], 0.0)
    v = jnp.dot(hid, wt2_ref[:], preferred_element_type=jnp.float32)
    w3 = wle_ref[:].reshape(1, 1, D_MAIN)
    b3 = ble_ref[:].reshape(1, 1, D_MAIN)
    v3 = v.reshape(rb, CTX, D_MAIN) + ys_ref[:][:, :, None] * w3 + b3
    ctxx = jnp.sum(probs[:, :, None] * v3, axis=1)
    x = x1_ref[:] + ctxx

    mu = x.mean(-1, keepdims=True)
    var = ((x - mu) ** 2).mean(-1, keepdims=True)
    h = (x - mu) / jnp.sqrt(var + 1e-5) * g1_ref[:] + bb1_ref[:]
    x = x + (jnp.maximum(
        jnp.dot(h, wb1_ref[:], preferred_element_type=jnp.float32) + b_b1_ref[:], 0.0)
        @ wb2_ref[:] + b_b2_ref[:])
    mu = x.mean(-1, keepdims=True)
    var = ((x - mu) ** 2).mean(-1, keepdims=True)
    h = (x - mu) / jnp.sqrt(var + 1e-5) * hg_ref[:] + hb_ref[:]
    out = jnp.dot(jnp.maximum(h, 0.0), wh_ref[:],
                  preferred_element_type=jnp.float32) + bh_ref[:]
    out_ref[:] = out


def _final(x1, k, ctx_rows, ysel, W_le, b_le, W_T1, b_T1, W_T2,
           ln1_g, ln1_b, W_b1, b_b1, W_b2, b_b2, h_g, h_b, W_h, b_h):
    RB = 128
    bh = x1.shape[0]
    r2 = lambda a: a.reshape(1, -1)
    return pl.pallas_call(
        _final_body,
        grid=(bh // RB,),
        in_specs=[
            pl.BlockSpec((RB, D_MAIN), lambda i: (i, 0)),
            pl.BlockSpec((RB, D_MAIN), lambda i: (i, 0)),
            pl.BlockSpec((RB * CTX, D_MAIN), lambda i: (i, 0)),
            pl.BlockSpec((RB, CTX), lambda i: (i, 0)),
            pl.BlockSpec((1, D_MAIN), lambda i: (0, 0)),
            pl.BlockSpec((1, D_MAIN), lambda i: (0, 0)),
            pl.BlockSpec((D_MAIN, D_BLOCK), lambda i: (0, 0)),
            pl.BlockSpec((1, D_BLOCK), lambda i: (0, 0)),
            pl.BlockSpec((D_BLOCK, D_MAIN), lambda i: (0, 0)),
            pl.BlockSpec((1, D_MAIN), lambda i: (0, 0)),
            pl.BlockSpec((1, D_MAIN), lambda i: (0, 0)),
            pl.BlockSpec((D_MAIN, D_BLOCK), lambda i: (0, 0)),
            pl.BlockSpec((1, D_BLOCK), lambda i: (0, 0)),
            pl.BlockSpec((D_BLOCK, D_MAIN), lambda i: (0, 0)),
            pl.BlockSpec((1, D_MAIN), lambda i: (0, 0)),
            pl.BlockSpec((1, D_MAIN), lambda i: (0, 0)),
            pl.BlockSpec((1, D_MAIN), lambda i: (0, 0)),
            pl.BlockSpec((D_MAIN, 1), lambda i: (0, 0)),
            pl.BlockSpec((1, 1), lambda i: (0, 0)),
        ],
        out_specs=pl.BlockSpec((RB, 1), lambda i: (i, 0)),
        out_shape=jax.ShapeDtypeStruct((bh, 1), jnp.float32),
    )(x1, k, ctx_rows, ysel, r2(W_le), r2(b_le), W_T1, r2(b_T1), W_T2,
      r2(ln1_g), r2(ln1_b), W_b1, r2(b_b1), W_b2, r2(b_b2), r2(h_g), r2(h_b),
      W_h, b_h.reshape(1, 1))


def kernel(x_num, candidate_x_num, candidate_y, W_in, b_in, W_K, b_K, W_le, b_le,
           W_T1, b_T1, W_T2, ln1_g, ln1_b, W_b1, b_b1, W_b2, b_b2, h_g, h_b, W_h, b_h,
           context_size):
    cand_p = jnp.pad(candidate_x_num, ((0, NC_PAD - N_CAND), (0, 0)))
    y_p = jnp.pad(candidate_y, (0, NC_PAD - N_CAND))
    bi = b_in.reshape(1, -1)
    bk = b_K.reshape(1, -1)

    ck_tbl, ck_aug = _encode_candidates(cand_p, W_in, bi, W_K, bk)
    x1, k, k_aug = _encode_queries(x_num, W_in, bi, W_K, bk)
    keys = _distance_keys(k_aug, ck_aug)

    outs = []
    H = B // 8
    for h in range(8):
        sl = slice(h * H, (h + 1) * H)
        keys_h = lax.slice(keys, (h * H, 0), ((h + 1) * H, NC_PAD))
        t_h = _threshold(keys_h)
        ctx_h, ysel_pad_h = _sc_select_gather(keys_h, t_h, ck_tbl, y_p)
        outs.append(_final(x1[sl], k[sl], ctx_h, ysel_pad_h[:, :CTX],
                           W_le, b_le, W_T1, b_T1, W_T2,
                           ln1_g, ln1_b, W_b1, b_b1, W_b2, b_b2,
                           h_g, h_b, W_h, b_h))
    return jnp.concatenate(outs, axis=0)
